# baseline (speedup 1.0000x reference)
import jax
import jax.numpy as jnp
from jax.experimental import pallas as pl


def _cell_kernel(x_ref, h_ref, c_ref, bd_ref, th_ref, bias_ref, wcv_ref,
                 lm_ref, linb_ref, out_ref, hout_ref, cout_ref):
    xb = x_ref[0]
    hb = h_ref[0]
    cb = c_ref[0]
    g4 = (jnp.dot(xb, bd_ref[...], preferred_element_type=jnp.float32)
          + jnp.dot(hb, th_ref[...], preferred_element_type=jnp.float32)
          + bias_ref[...])
    wcv = wcv_ref[...]
    gi = jax.nn.sigmoid(g4[:, 0:96] + wcv[0:1, :] * cb)
    gf = jax.nn.sigmoid(g4[:, 128:224] + wcv[1:2, :] * cb)
    gt = jnp.tanh(g4[:, 256:352])
    c_new = gf * cb + gi * gt
    go = jax.nn.sigmoid(g4[:, 384:480] + wcv[2:3, :] * c_new)
    h_new = go * jnp.tanh(c_new)
    out_ref[0] = (jnp.dot(jax.nn.relu(h_new), lm_ref[...],
                          preferred_element_type=jnp.float32)
                  + linb_ref[...])
    hout_ref[0] = h_new
    cout_ref[0] = c_new


def kernel(x, edge_index, edge_weight, h, c,
           W_i, W_f, W_c, W_o,
           Th_i, Th_f, Th_c, Th_o,
           cb_i, cb_f, cb_c, cb_o,
           b_i, b_f, b_c, b_o,
           wc_i, wc_f, wc_o,
           lin_W, lin_b):
    n = x.shape[0]
    xf = x.reshape(25, 125, 384)
    hf = h.reshape(25, 125, 96)
    cf = c.reshape(25, 125, 96)

    eye32 = jnp.eye(32, dtype=jnp.float32)

    def gate_block(w):
        bd = jnp.kron(eye32, w)
        return jnp.pad(bd, ((0, 0), (0, 32)))

    bd = jnp.concatenate([gate_block(w) for w in (W_i, W_f, W_c, W_o)],
                         axis=1)
    th = jnp.concatenate([gate_block(t) for t in (Th_i, Th_f, Th_c, Th_o)],
                         axis=1)
    bias_row = jnp.concatenate(
        [jnp.pad(jnp.tile(cbg[None, :] + bg, (1, 32)), ((0, 0), (0, 32)))
         for cbg, bg in ((cb_i, b_i), (cb_f, b_f), (cb_c, b_c), (cb_o, b_o))],
        axis=1)
    wcv = jnp.concatenate([jnp.tile(w, (1, 32)) for w in (wc_i, wc_f, wc_o)],
                          axis=0)
    lm = jnp.kron(eye32, lin_W.T)
    linb = lin_b.reshape(1, 1)

    row3 = lambda w: pl.BlockSpec((1, 125, w), lambda i: (i, 0, 0))
    full = lambda a: pl.BlockSpec(a.shape, lambda i: (0,) * a.ndim)

    outf, hof, cof = pl.pallas_call(
        _cell_kernel,
        grid=(25,),
        in_specs=[
            row3(384), row3(96), row3(96),
            full(bd), full(th), full(bias_row), full(wcv),
            full(lm), full(linb),
        ],
        out_specs=[row3(32), row3(96), row3(96)],
        out_shape=[
            jax.ShapeDtypeStruct((25, 125, 32), jnp.float32),
            jax.ShapeDtypeStruct((25, 125, 96), jnp.float32),
            jax.ShapeDtypeStruct((25, 125, 96), jnp.float32),
        ],
    )(xf, hf, cf, bd, th, bias_row, wcv, lm, linb)
    return (outf.reshape(n, 1), hof.reshape(n, 3), cof.reshape(n, 3))


# speedup vs baseline: 12.5997x; 12.5997x over previous
import jax
import jax.numpy as jnp
from jax.experimental import pallas as pl

_L = 6400

_DN = (((0,), (0,)), ((), ()))


def _cell_kernel(x_ref, hc_ref, w_ref, th_ref, wcs_ref, linw_ref, linb_ref,
                 out_ref, hout_ref, cout_ref):
    xb = x_ref[...]
    hc = hc_ref[...]
    cb = hc[3:6, :]
    wcs = wcs_ref[...]

    def gate(g):
        zx = jax.lax.dot_general(w_ref[:, 3 * g:3 * g + 3], xb, _DN,
                                 preferred_element_type=jnp.float32)
        zh = jax.lax.dot_general(th_ref[:, 3 * g:3 * g + 3], hc, _DN,
                                 preferred_element_type=jnp.float32)
        return zx + zh

    gi = jax.nn.sigmoid(gate(0) + wcs[:, 0:1] * cb)
    gf = jax.nn.sigmoid(gate(1) + wcs[:, 1:2] * cb)
    gt = jnp.tanh(gate(2))
    c_new = gf * cb + gi * gt
    go = jax.nn.sigmoid(gate(3) + wcs[:, 2:3] * c_new)
    h_new = go * jnp.tanh(c_new)
    out_ref[...] = (jax.lax.dot_general(
        linw_ref[...], jax.nn.relu(h_new), (((1,), (0,)), ((), ())),
        preferred_element_type=jnp.float32) + linb_ref[...])
    hout_ref[...] = h_new
    cout_ref[...] = c_new


def kernel(x, edge_index, edge_weight, h, c,
           W_i, W_f, W_c, W_o,
           Th_i, Th_f, Th_c, Th_o,
           cb_i, cb_f, cb_c, cb_o,
           b_i, b_f, b_c, b_o,
           wc_i, wc_f, wc_o,
           lin_W, lin_b):
    n = x.shape[0]
    xt = x.T
    hcb = jnp.concatenate(
        [h, c, jnp.ones((n, 1), jnp.float32)], axis=1).T
    wcat = jnp.concatenate([W_i, W_f, W_c, W_o], axis=1)
    th_aug = jnp.concatenate([
        jnp.concatenate([Th_i, Th_f, Th_c, Th_o], axis=1),
        jnp.zeros((3, 12), jnp.float32),
        jnp.concatenate([cb_i[None, :] + b_i, cb_f[None, :] + b_f,
                         cb_c[None, :] + b_c, cb_o[None, :] + b_o], axis=1),
    ], axis=0)
    wcs = jnp.stack([wc_i[0], wc_f[0], wc_o[0]], axis=1)
    linb = lin_b.reshape(1, 1)

    grid = (pl.cdiv(n, _L),)
    lane = lambda r: pl.BlockSpec((r, _L), lambda i: (0, i))
    full = lambda a: pl.BlockSpec(a.shape, lambda i: (0, 0))

    outt, ht, ct = pl.pallas_call(
        _cell_kernel,
        grid=grid,
        in_specs=[
            lane(12), lane(7),
            full(wcat), full(th_aug), full(wcs), full(lin_W), full(linb),
        ],
        out_specs=[lane(1), lane(3), lane(3)],
        out_shape=[
            jax.ShapeDtypeStruct((1, n), jnp.float32),
            jax.ShapeDtypeStruct((3, n), jnp.float32),
            jax.ShapeDtypeStruct((3, n), jnp.float32),
        ],
    )(xt, hcb, wcat, th_aug, wcs, lin_W, linb)
    return (outt.T, ht.T, ct.T)


# speedup vs baseline: 14.7633x; 1.1717x over previous
import jax
import jax.numpy as jnp
from jax.experimental import pallas as pl

_L = 12800

_MM = (((1,), (0,)), ((), ()))
_CC = (((0,), (0,)), ((), ()))


def _sig(z):
    return 0.5 * jnp.tanh(0.5 * z) + 0.5


def _cell_kernel(x_ref, hc_ref, w_ref, out_ref, hout_ref, cout_ref):
    xb = x_ref[...]
    hc = hc_ref[...]
    w = w_ref[...]
    cb = hc[3:6, :]

    def gate(g):
        zx = jax.lax.dot_general(w[3 * g:3 * g + 3, 0:12], xb, _MM,
                                 preferred_element_type=jnp.float32)
        zh = jax.lax.dot_general(w[3 * g:3 * g + 3, 12:19], hc, _MM,
                                 preferred_element_type=jnp.float32)
        return zx + zh

    gi = _sig(gate(0) + w[0:3, 19:20] * cb)
    gf = _sig(gate(1) + w[0:3, 20:21] * cb)
    gt = jnp.tanh(gate(2))
    c_new = gf * cb + gi * gt
    go = _sig(gate(3) + w[0:3, 21:22] * c_new)
    h_new = go * jnp.tanh(c_new)
    out_ref[...] = (jax.lax.dot_general(
        w[0:3, 22:23], jax.nn.relu(h_new), _CC,
        preferred_element_type=jnp.float32) + w[0:1, 23:24])
    hout_ref[...] = h_new
    cout_ref[...] = c_new


def kernel(x, edge_index, edge_weight, h, c,
           W_i, W_f, W_c, W_o,
           Th_i, Th_f, Th_c, Th_o,
           cb_i, cb_f, cb_c, cb_o,
           b_i, b_f, b_c, b_o,
           wc_i, wc_f, wc_o,
           lin_W, lin_b):
    n = x.shape[0]
    f32 = jnp.float32
    xt = x.T
    hcb = jnp.concatenate(
        [h, c, jnp.ones((n, 1), f32)], axis=1).T

    z33 = jnp.zeros((3, 3), f32)
    th_big = jnp.concatenate(
        [jnp.concatenate([tg.T, z33, cbg[:, None] + bg.T], axis=1)
         for tg, cbg, bg in ((Th_i, cb_i, b_i), (Th_f, cb_f, b_f),
                             (Th_c, cb_c, b_c), (Th_o, cb_o, b_o))],
        axis=0)
    wcat_t = jnp.concatenate([W_i.T, W_f.T, W_c.T, W_o.T], axis=0)
    pad9 = lambda a: jnp.pad(a, ((0, 12 - a.shape[0]), (0, 0)))
    wcs = pad9(jnp.concatenate([wc_i.T, wc_f.T, wc_o.T], axis=1))
    lin_col = pad9(lin_W.T)
    linb_col = pad9(lin_b.reshape(1, 1))
    w_all = jnp.concatenate([wcat_t, th_big, wcs, lin_col, linb_col],
                            axis=1)

    grid = (pl.cdiv(n, _L),)
    lane = lambda r: pl.BlockSpec((r, _L), lambda i: (0, i))

    outt, ht, ct = pl.pallas_call(
        _cell_kernel,
        grid=grid,
        in_specs=[lane(12), lane(7),
                  pl.BlockSpec((12, 24), lambda i: (0, 0))],
        out_specs=[lane(1), lane(3), lane(3)],
        out_shape=[
            jax.ShapeDtypeStruct((1, n), f32),
            jax.ShapeDtypeStruct((3, n), f32),
            jax.ShapeDtypeStruct((3, n), f32),
        ],
    )(xt, hcb, w_all)
    return (outt.T, ht.T, ct.T)


# speedup vs baseline: 14.7937x; 1.0021x over previous
import jax
import jax.numpy as jnp
from jax.experimental import pallas as pl

_L = 12800

_MM = (((1,), (0,)), ((), ()))
_CC = (((0,), (0,)), ((), ()))


def _sig(z):
    return 0.5 * jnp.tanh(0.5 * z) + 0.5


def _cell_kernel(x_ref, hc_ref, w_ref, out_ref, hout_ref, cout_ref):
    xb = x_ref[...]
    hc = hc_ref[...]
    w = w_ref[...]
    cb = hc[3:6, :]

    def gate(g):
        zx = jax.lax.dot_general(w[3 * g:3 * g + 3, 0:12], xb, _MM,
                                 preferred_element_type=jnp.float32)
        zh = jax.lax.dot_general(w[3 * g:3 * g + 3, 12:19], hc, _MM,
                                 preferred_element_type=jnp.float32)
        return zx + zh

    gi = _sig(gate(0) + w[0:3, 19:20] * cb)
    gf = _sig(gate(1) + w[0:3, 20:21] * cb)
    gt = jnp.tanh(gate(2))
    c_new = gf * cb + gi * gt
    go = _sig(gate(3) + w[0:3, 21:22] * c_new)
    h_new = go * jnp.tanh(c_new)
    out_ref[...] = (jax.lax.dot_general(
        w[0:3, 22:23], jax.nn.relu(h_new), _CC,
        preferred_element_type=jnp.float32) + w[0:1, 23:24])
    hout_ref[...] = h_new
    cout_ref[...] = c_new


def kernel(x, edge_index, edge_weight, h, c,
           W_i, W_f, W_c, W_o,
           Th_i, Th_f, Th_c, Th_o,
           cb_i, cb_f, cb_c, cb_o,
           b_i, b_f, b_c, b_o,
           wc_i, wc_f, wc_o,
           lin_W, lin_b):
    n = x.shape[0]
    f32 = jnp.float32
    xt = x.T
    hcb = jnp.concatenate(
        [h, c, jnp.ones((n, 1), f32)], axis=1).T

    def put(a, r0, c0):
        return jnp.pad(a, ((r0, 12 - r0 - a.shape[0]),
                           (c0, 24 - c0 - a.shape[1])))

    pieces = []
    gates = ((W_i, Th_i, cb_i, b_i), (W_f, Th_f, cb_f, b_f),
             (W_c, Th_c, cb_c, b_c), (W_o, Th_o, cb_o, b_o))
    for g, (wg, tg, cbg, bg) in enumerate(gates):
        pieces.append(put(wg.T, 3 * g, 0))
        pieces.append(put(tg.T, 3 * g, 12))
        pieces.append(put(cbg[:, None] + bg.T, 3 * g, 18))
    for g, wcg in enumerate((wc_i, wc_f, wc_o)):
        pieces.append(put(wcg.T, 0, 19 + g))
    pieces.append(put(lin_W.T, 0, 22))
    pieces.append(put(lin_b.reshape(1, 1), 0, 23))
    w_all = sum(pieces)

    grid = (pl.cdiv(n, _L),)
    lane = lambda r: pl.BlockSpec((r, _L), lambda i: (0, i))

    outt, ht, ct = pl.pallas_call(
        _cell_kernel,
        grid=grid,
        in_specs=[lane(12), lane(7),
                  pl.BlockSpec((12, 24), lambda i: (0, 0))],
        out_specs=[lane(1), lane(3), lane(3)],
        out_shape=[
            jax.ShapeDtypeStruct((1, n), f32),
            jax.ShapeDtypeStruct((3, n), f32),
            jax.ShapeDtypeStruct((3, n), f32),
        ],
    )(xt, hcb, w_all)
    return (outt.T, ht.T, ct.T)


# speedup vs baseline: 16.3421x; 1.1047x over previous
import jax
import jax.numpy as jnp
from jax.experimental import pallas as pl

_L = 12800

_CC = (((0,), (0,)), ((), ()))
_MM = (((1,), (0,)), ((), ()))


def _sig(z):
    return 0.5 * jnp.tanh(0.5 * z) + 0.5


def _cell_kernel(x_ref, hc_ref, w_ref, out_ref, hout_ref, cout_ref):
    xb = x_ref[...]
    hc = hc_ref[...]
    w = w_ref[...]
    cb = hc[3:6, :]

    def zgate(g):
        zx = jax.lax.dot_general(w[:, 3 * g:3 * g + 3], xb, _CC,
                                 preferred_element_type=jnp.float32)
        zh = jax.lax.dot_general(w[0:7, 12 + 3 * g:15 + 3 * g], hc, _CC,
                                 preferred_element_type=jnp.float32)
        return zx + zh

    gi = _sig(zgate(0))
    gf = _sig(zgate(1))
    gt = jnp.tanh(zgate(2))
    c_new = gf * cb + gi * gt
    zo = zgate(3) + jax.lax.dot_general(w[0:3, 24:27], c_new, _CC,
                                        preferred_element_type=jnp.float32)
    go = _sig(zo)
    h_new = go * jnp.tanh(c_new)
    out_ref[...] = (jax.lax.dot_general(
        w[4:5, 24:27], jax.nn.relu(h_new), _MM,
        preferred_element_type=jnp.float32) + w[5:6, 24:25])
    hout_ref[...] = h_new
    cout_ref[...] = c_new


def kernel(x, edge_index, edge_weight, h, c,
           W_i, W_f, W_c, W_o,
           Th_i, Th_f, Th_c, Th_o,
           cb_i, cb_f, cb_c, cb_o,
           b_i, b_f, b_c, b_o,
           wc_i, wc_f, wc_o,
           lin_W, lin_b):
    n = x.shape[0]
    f32 = jnp.float32
    xt = x.T
    hcb = jnp.concatenate(
        [h, c, jnp.ones((n, 1), f32)], axis=1).T

    def put(a, r0, c0):
        return jnp.pad(a, ((r0, 12 - r0 - a.shape[0]),
                           (c0, 28 - c0 - a.shape[1])))

    eye3 = jnp.eye(3, dtype=f32)
    gates = ((W_i, Th_i, cb_i, b_i), (W_f, Th_f, cb_f, b_f),
             (W_c, Th_c, cb_c, b_c), (W_o, Th_o, cb_o, b_o))
    pieces = []
    for g, (wg, tg, cbg, bg) in enumerate(gates):
        pieces.append(put(wg, 0, 3 * g))
        pieces.append(put(tg, 0, 12 + 3 * g))
        pieces.append(put(cbg[None, :] + bg, 6, 12 + 3 * g))
    pieces.append(put(wc_i * eye3, 3, 12))
    pieces.append(put(wc_f * eye3, 3, 15))
    pieces.append(put(wc_o * eye3, 0, 24))
    pieces.append(put(lin_W, 4, 24))
    pieces.append(put(lin_b.reshape(1, 1), 5, 24))
    w_all = sum(pieces)

    grid = (pl.cdiv(n, _L),)
    lane = lambda r: pl.BlockSpec((r, _L), lambda i: (0, i))

    outt, ht, ct = pl.pallas_call(
        _cell_kernel,
        grid=grid,
        in_specs=[lane(12), lane(7),
                  pl.BlockSpec((12, 28), lambda i: (0, 0))],
        out_specs=[lane(1), lane(3), lane(3)],
        out_shape=[
            jax.ShapeDtypeStruct((1, n), f32),
            jax.ShapeDtypeStruct((3, n), f32),
            jax.ShapeDtypeStruct((3, n), f32),
        ],
    )(xt, hcb, w_all)
    return (outt.T, ht.T, ct.T)
